# Initial kernel scaffold; baseline (speedup 1.0000x reference)
import jax
import jax.numpy as jnp
from jax.experimental import pallas as pl


def kernel(hidden_states, W1, b1, W2, b2, gate_w, U, alpha, w_gate, w_up, w_down, gumbel):
    raise NotImplementedError("write your pallas kernel here")


# speedup vs baseline: 1.0829x; 1.0829x over previous
import functools

import jax
import jax.numpy as jnp
from jax import lax
from jax.experimental import pallas as pl
from jax.experimental.pallas import tpu as pltpu
from jax.experimental.pallas import tpu_sc as plsc

_E = 64
_NZ = 8
_TILE = 8
_P = 768
_NC = 2
_NS = 16
_NW = _NC * _NS


def _router_body(x_ref, w1_ref, b1_ref, w2_ref, b2_ref, gw_ref, u_ref,
                 alpha_ref, gum_ref, pos_ref, inv_ref, starts_ref,
                 ntiles_ref, rws_ref):
    f32 = jnp.float32
    x = x_ref[...]
    T = x.shape[0]

    a1 = lax.dot_general(x, w1_ref[...], (((1,), (1,)), ((), ()))) + b1_ref[...]
    h = a1 / (1.0 + jnp.exp(-a1))
    zlog = lax.dot_general(h, w2_ref[...], (((1,), (1,)), ((), ()))) + b2_ref[...]

    g = zlog + gum_ref[...]
    gmax = jnp.max(g, axis=1, keepdims=True)
    iotz = lax.broadcasted_iota(jnp.int32, (T, _NZ), 1)
    zidx = jnp.min(jnp.where(g == gmax, iotz, _NZ), axis=1, keepdims=True)
    zoh = (iotz == zidx).astype(f32)
    zbias = lax.dot_general(zoh, u_ref[...], (((1,), (0,)), ((), ())))

    logits = lax.dot_general(x, gw_ref[...], (((1,), (1,)), ((), ())))
    logits = logits + alpha_ref[0, 0] * zbias
    lmax = jnp.max(logits, axis=1, keepdims=True)
    el = jnp.exp(logits - lmax)
    probs = el / jnp.sum(el, axis=1, keepdims=True)
    pmax = jnp.max(probs, axis=1, keepdims=True)
    iote = lax.broadcasted_iota(jnp.int32, (T, _E), 1)
    sel = jnp.min(jnp.where(probs == pmax, iote, _E), axis=1, keepdims=True)
    sel_oh = (iote == sel).astype(f32)

    ones_row = jnp.ones((1, T), f32)
    counts = lax.dot_general(ones_row, sel_oh, (((1,), (0,)), ((), ())))
    ptiles = jnp.floor((counts + (_TILE - 1)) * (1.0 / _TILE))
    pc = ptiles * _TILE
    ia = lax.broadcasted_iota(jnp.int32, (_E, _E), 0)
    ib = lax.broadcasted_iota(jnp.int32, (_E, _E), 1)
    tri = (ia < ib).astype(f32)
    pstarts = lax.dot_general(pc, tri, (((1,), (0,)), ((), ())))

    eqf = lax.dot_general(sel_oh, sel_oh, (((1,), (1,)), ((), ())))
    it0 = lax.broadcasted_iota(jnp.int32, (T, T), 0)
    it1 = lax.broadcasted_iota(jnp.int32, (T, T), 1)
    lt = (it1 < it0).astype(f32)
    rank_col = jnp.sum(eqf * lt, axis=1, keepdims=True)

    pstart_sel = lax.dot_general(sel_oh, pstarts, (((1,), (1,)), ((), ())))
    pos_col = pstart_sel + rank_col
    eye = (it0 == it1).astype(f32)
    pos_row = lax.dot_general(ones_row, eye * pos_col, (((1,), (0,)), ((), ())))

    mp = (lax.broadcasted_iota(jnp.int32, (_P, T), 0).astype(f32)
          == pos_row).astype(f32)
    ar_col = lax.broadcasted_iota(jnp.int32, (T, 1), 0).astype(f32)
    inv_col = lax.dot_general(mp, ar_col, (((1,), (0,)), ((), ())))
    rws_col = lax.dot_general(mp, pmax, (((1,), (0,)), ((), ())))

    pos_ref[...] = pos_row.astype(jnp.int32)
    inv_ref[...] = inv_col.astype(jnp.int32)
    starts_ref[...] = pstarts.astype(jnp.int32)
    ntiles_ref[...] = ptiles.astype(jnp.int32)
    rws_ref[...] = rws_col


def _router(flat, W1, b1r, W2, b2r, gate_w, U, alpha_r, gumbel):
    T = flat.shape[0]
    return pl.pallas_call(
        _router_body,
        out_shape=(
            jax.ShapeDtypeStruct((1, T), jnp.int32),
            jax.ShapeDtypeStruct((_P, 1), jnp.int32),
            jax.ShapeDtypeStruct((1, _E), jnp.int32),
            jax.ShapeDtypeStruct((1, _E), jnp.int32),
            jax.ShapeDtypeStruct((_P, 1), jnp.float32),
        ),
        in_specs=[
            pl.BlockSpec(memory_space=pltpu.VMEM),
            pl.BlockSpec(memory_space=pltpu.VMEM),
            pl.BlockSpec(memory_space=pltpu.VMEM),
            pl.BlockSpec(memory_space=pltpu.VMEM),
            pl.BlockSpec(memory_space=pltpu.VMEM),
            pl.BlockSpec(memory_space=pltpu.VMEM),
            pl.BlockSpec(memory_space=pltpu.VMEM),
            pl.BlockSpec(memory_space=pltpu.SMEM),
            pl.BlockSpec(memory_space=pltpu.VMEM),
        ],
        out_specs=(
            pl.BlockSpec(memory_space=pltpu.VMEM),
            pl.BlockSpec(memory_space=pltpu.VMEM),
            pl.BlockSpec(memory_space=pltpu.VMEM),
            pl.BlockSpec(memory_space=pltpu.VMEM),
            pl.BlockSpec(memory_space=pltpu.VMEM),
        ),
    )(flat, W1, b1r, W2, b2r, gate_w, U, alpha_r, gumbel)


def _sc_gather(idx, table, n_rows):
    d = table.shape[1]
    rpt = n_rows // _NW
    mesh = plsc.VectorSubcoreMesh(core_axis_name="c", subcore_axis_name="s")

    @functools.partial(
        pl.kernel,
        out_type=jax.ShapeDtypeStruct((n_rows, d), table.dtype),
        mesh=mesh,
        scratch_types=[
            pltpu.VMEM((rpt,), jnp.int32),
            pltpu.VMEM((rpt, d), table.dtype),
            pltpu.SemaphoreType.DMA,
        ],
    )
    def gather_k(idx_hbm, table_hbm, out_hbm, idx_v, rows_v, sem):
        wid = lax.axis_index("s") * _NC + lax.axis_index("c")
        base = wid * rpt
        pltpu.sync_copy(idx_hbm.at[pl.ds(base, rpt)], idx_v)
        pltpu.async_copy(table_hbm.at[idx_v], rows_v, sem).wait()
        pltpu.sync_copy(rows_v, out_hbm.at[pl.ds(base, rpt)])

    return gather_k(idx, table)


def _expert_body(starts_ref, ntiles_ref, xs_ref, wg_ref, wu_ref, wd_ref,
                 rws_ref, out_ref):
    e = pl.program_id(0)
    start = starts_ref[0, e]
    nt = ntiles_ref[0, e]
    wg = wg_ref[0]
    wu = wu_ref[0]
    wd = wd_ref[0]

    def tile_body(i, carry):
        off = pl.multiple_of(start + i * _TILE, _TILE)
        x8 = xs_ref[pl.ds(off, _TILE), :]
        gg = lax.dot_general(x8, wg, (((1,), (1,)), ((), ())))
        uu = lax.dot_general(x8, wu, (((1,), (1,)), ((), ())))
        hh = gg / (1.0 + jnp.exp(-gg)) * uu
        yy = lax.dot_general(hh, wd, (((1,), (1,)), ((), ())))
        out_ref[pl.ds(off, _TILE), :] = yy * rws_ref[pl.ds(off, _TILE), :]
        return carry

    lax.fori_loop(0, nt, tile_body, 0)


def _experts(xs, w_gate, w_up, w_down, rws, pstarts, ntiles):
    dff, d = w_gate.shape[1], w_gate.shape[2]
    return pl.pallas_call(
        _expert_body,
        grid=(_E,),
        out_shape=jax.ShapeDtypeStruct((_P, d), jnp.float32),
        in_specs=[
            pl.BlockSpec(memory_space=pltpu.SMEM),
            pl.BlockSpec(memory_space=pltpu.SMEM),
            pl.BlockSpec((_P, d), lambda e: (0, 0)),
            pl.BlockSpec((1, dff, d), lambda e: (e, 0, 0)),
            pl.BlockSpec((1, dff, d), lambda e: (e, 0, 0)),
            pl.BlockSpec((1, d, dff), lambda e: (e, 0, 0)),
            pl.BlockSpec((_P, 1), lambda e: (0, 0)),
        ],
        out_specs=pl.BlockSpec((_P, d), lambda e: (0, 0)),
        compiler_params=pltpu.CompilerParams(
            dimension_semantics=("arbitrary",),
        ),
    )(pstarts, ntiles, xs, w_gate, w_up, w_down, rws)


def kernel(hidden_states, W1, b1, W2, b2, gate_w, U, alpha, w_gate, w_up,
           w_down, gumbel):
    bq, sq, d = hidden_states.shape
    flat = hidden_states.reshape(-1, d)
    b1r = b1.reshape(1, -1)
    b2r = b2.reshape(1, -1)
    alpha_r = jnp.asarray(alpha, jnp.float32).reshape(1, 1)

    pos, inv, pstarts, ntiles, rws = _router(
        flat, W1, b1r, W2, b2r, gate_w, U, alpha_r, gumbel)
    xs = _sc_gather(inv.reshape(-1), flat, _P)
    out_sorted = _experts(xs, w_gate, w_up, w_down, rws, pstarts, ntiles)
    out = _sc_gather(pos.reshape(-1), out_sorted, flat.shape[0])
    return out.reshape(bq, sq, d)


# speedup vs baseline: 1.0851x; 1.0021x over previous
import functools

import jax
import jax.numpy as jnp
from jax import lax
from jax.experimental import pallas as pl
from jax.experimental.pallas import tpu as pltpu
from jax.experimental.pallas import tpu_sc as plsc

_E = 64
_NZ = 8
_TILE = 8
_P = 768
_NC = 2
_NS = 16
_NW = _NC * _NS


def _router_body(x_ref, w1_ref, b1_ref, w2_ref, b2_ref, gw_ref, u_ref,
                 alpha_ref, gum_ref, pos_ref, inv_ref, starts_ref,
                 ntiles_ref, rws_ref):
    f32 = jnp.float32
    x = x_ref[...]
    T = x.shape[0]

    a1 = lax.dot_general(x, w1_ref[...], (((1,), (1,)), ((), ()))) + b1_ref[...]
    h = a1 / (1.0 + jnp.exp(-a1))
    zlog = lax.dot_general(h, w2_ref[...], (((1,), (1,)), ((), ()))) + b2_ref[...]

    g = zlog + gum_ref[...]
    gmax = jnp.max(g, axis=1, keepdims=True)
    iotz = lax.broadcasted_iota(jnp.int32, (T, _NZ), 1)
    zidx = jnp.min(jnp.where(g == gmax, iotz, _NZ), axis=1, keepdims=True)
    zoh = (iotz == zidx).astype(f32)
    zbias = lax.dot_general(zoh, u_ref[...], (((1,), (0,)), ((), ())))

    logits = lax.dot_general(x, gw_ref[...], (((1,), (1,)), ((), ())))
    logits = logits + alpha_ref[0, 0] * zbias
    lmax = jnp.max(logits, axis=1, keepdims=True)
    el = jnp.exp(logits - lmax)
    probs = el / jnp.sum(el, axis=1, keepdims=True)
    pmax = jnp.max(probs, axis=1, keepdims=True)
    iote = lax.broadcasted_iota(jnp.int32, (T, _E), 1)
    sel = jnp.min(jnp.where(probs == pmax, iote, _E), axis=1, keepdims=True)
    sel_oh = (iote == sel).astype(f32)

    ones_row = jnp.ones((1, T), f32)
    counts = lax.dot_general(ones_row, sel_oh, (((1,), (0,)), ((), ())))
    ptiles = jnp.floor((counts + (_TILE - 1)) * (1.0 / _TILE))
    pc = ptiles * _TILE
    ia = lax.broadcasted_iota(jnp.int32, (_E, _E), 0)
    ib = lax.broadcasted_iota(jnp.int32, (_E, _E), 1)
    tri = (ia < ib).astype(f32)
    pstarts = lax.dot_general(pc, tri, (((1,), (0,)), ((), ())))

    eqf = lax.dot_general(sel_oh, sel_oh, (((1,), (1,)), ((), ())))
    it0 = lax.broadcasted_iota(jnp.int32, (T, T), 0)
    it1 = lax.broadcasted_iota(jnp.int32, (T, T), 1)
    lt = (it1 < it0).astype(f32)
    rank_col = jnp.sum(eqf * lt, axis=1, keepdims=True)

    pstart_sel = lax.dot_general(sel_oh, pstarts, (((1,), (1,)), ((), ())))
    pos_col = pstart_sel + rank_col
    eye = (it0 == it1).astype(f32)
    pos_row = lax.dot_general(ones_row, eye * pos_col, (((1,), (0,)), ((), ())))

    mp = (lax.broadcasted_iota(jnp.int32, (_P, T), 0).astype(f32)
          == pos_row).astype(f32)
    ar_col = lax.broadcasted_iota(jnp.int32, (T, 1), 0).astype(f32)
    inv_col = lax.dot_general(mp, ar_col, (((1,), (0,)), ((), ())))
    rws_col = lax.dot_general(mp, pmax, (((1,), (0,)), ((), ())))

    pos_ref[...] = pos_row.astype(jnp.int32)
    inv_ref[...] = inv_col.astype(jnp.int32)
    starts_ref[...] = pstarts.astype(jnp.int32)
    ntiles_ref[...] = ptiles.astype(jnp.int32)
    rws_ref[...] = rws_col


def _router(flat, W1, b1r, W2, b2r, gate_w, U, alpha_r, gumbel):
    T = flat.shape[0]
    return pl.pallas_call(
        _router_body,
        out_shape=(
            jax.ShapeDtypeStruct((1, T), jnp.int32),
            jax.ShapeDtypeStruct((_P, 1), jnp.int32),
            jax.ShapeDtypeStruct((1, _E), jnp.int32),
            jax.ShapeDtypeStruct((1, _E), jnp.int32),
            jax.ShapeDtypeStruct((_P, 1), jnp.float32),
        ),
        in_specs=[
            pl.BlockSpec(memory_space=pltpu.VMEM),
            pl.BlockSpec(memory_space=pltpu.VMEM),
            pl.BlockSpec(memory_space=pltpu.VMEM),
            pl.BlockSpec(memory_space=pltpu.VMEM),
            pl.BlockSpec(memory_space=pltpu.VMEM),
            pl.BlockSpec(memory_space=pltpu.VMEM),
            pl.BlockSpec(memory_space=pltpu.VMEM),
            pl.BlockSpec(memory_space=pltpu.SMEM),
            pl.BlockSpec(memory_space=pltpu.VMEM),
        ],
        out_specs=(
            pl.BlockSpec(memory_space=pltpu.VMEM),
            pl.BlockSpec(memory_space=pltpu.VMEM),
            pl.BlockSpec(memory_space=pltpu.VMEM),
            pl.BlockSpec(memory_space=pltpu.VMEM),
            pl.BlockSpec(memory_space=pltpu.VMEM),
        ),
    )(flat, W1, b1r, W2, b2r, gate_w, U, alpha_r, gumbel)


def _sc_gather(idx, table, n_rows):
    d = table.shape[1]
    rpt = n_rows // _NW
    mesh = plsc.VectorSubcoreMesh(core_axis_name="c", subcore_axis_name="s")

    @functools.partial(
        pl.kernel,
        out_type=jax.ShapeDtypeStruct((n_rows, d), table.dtype),
        mesh=mesh,
        scratch_types=[
            pltpu.VMEM((rpt,), jnp.int32),
            pltpu.VMEM((rpt, d), table.dtype),
            pltpu.SemaphoreType.DMA,
        ],
    )
    def gather_k(idx_hbm, table_hbm, out_hbm, idx_v, rows_v, sem):
        wid = lax.axis_index("s") * _NC + lax.axis_index("c")
        base = wid * rpt
        pltpu.sync_copy(idx_hbm.at[pl.ds(base, rpt)], idx_v)
        copies = [
            pltpu.async_copy(
                table_hbm.at[idx_v.at[pl.ds(j * 8, 8)]],
                rows_v.at[pl.ds(j * 8, 8)], sem)
            for j in range(rpt // 8)
        ]
        for c in copies:
            c.wait()
        pltpu.sync_copy(rows_v, out_hbm.at[pl.ds(base, rpt)])

    return gather_k(idx, table)


def _expert_body(starts_ref, ntiles_ref, xs_ref, wg_ref, wu_ref, wd_ref,
                 rws_ref, out_ref):
    e = pl.program_id(0)
    start = starts_ref[0, e]
    nt = ntiles_ref[0, e]
    wg = wg_ref[0]
    wu = wu_ref[0]
    wd = wd_ref[0]

    def tile_body(i, carry):
        off = pl.multiple_of(start + i * _TILE, _TILE)
        x8 = xs_ref[pl.ds(off, _TILE), :]
        gg = lax.dot_general(x8, wg, (((1,), (1,)), ((), ())))
        uu = lax.dot_general(x8, wu, (((1,), (1,)), ((), ())))
        hh = gg / (1.0 + jnp.exp(-gg)) * uu
        yy = lax.dot_general(hh, wd, (((1,), (1,)), ((), ())))
        out_ref[pl.ds(off, _TILE), :] = yy * rws_ref[pl.ds(off, _TILE), :]
        return carry

    lax.fori_loop(0, nt, tile_body, 0)


def _experts(xs, w_gate, w_up, w_down, rws, pstarts, ntiles):
    dff, d = w_gate.shape[1], w_gate.shape[2]
    return pl.pallas_call(
        _expert_body,
        grid=(_E,),
        out_shape=jax.ShapeDtypeStruct((_P, d), jnp.float32),
        in_specs=[
            pl.BlockSpec(memory_space=pltpu.SMEM),
            pl.BlockSpec(memory_space=pltpu.SMEM),
            pl.BlockSpec((_P, d), lambda e: (0, 0)),
            pl.BlockSpec((1, dff, d), lambda e: (e, 0, 0)),
            pl.BlockSpec((1, dff, d), lambda e: (e, 0, 0)),
            pl.BlockSpec((1, d, dff), lambda e: (e, 0, 0)),
            pl.BlockSpec((_P, 1), lambda e: (0, 0)),
        ],
        out_specs=pl.BlockSpec((_P, d), lambda e: (0, 0)),
        compiler_params=pltpu.CompilerParams(
            dimension_semantics=("arbitrary",),
        ),
    )(pstarts, ntiles, xs, w_gate, w_up, w_down, rws)


def kernel(hidden_states, W1, b1, W2, b2, gate_w, U, alpha, w_gate, w_up,
           w_down, gumbel):
    bq, sq, d = hidden_states.shape
    flat = hidden_states.reshape(-1, d)
    b1r = b1.reshape(1, -1)
    b2r = b2.reshape(1, -1)
    alpha_r = jnp.asarray(alpha, jnp.float32).reshape(1, 1)

    pos, inv, pstarts, ntiles, rws = _router(
        flat, W1, b1r, W2, b2r, gate_w, U, alpha_r, gumbel)
    xs = _sc_gather(inv.reshape(-1), flat, _P)
    out_sorted = _experts(xs, w_gate, w_up, w_down, rws, pstarts, ntiles)
    out = _sc_gather(pos.reshape(-1), out_sorted, flat.shape[0])
    return out.reshape(bq, sq, d)


# speedup vs baseline: 1.2256x; 1.1294x over previous
import functools

import jax
import jax.numpy as jnp
from jax import lax
from jax.experimental import pallas as pl
from jax.experimental.pallas import tpu as pltpu
from jax.experimental.pallas import tpu_sc as plsc

_E = 64
_NZ = 8
_TILE = 8
_P = 768
_NC = 2
_NS = 16
_NW = _NC * _NS


def _router_body(x_ref, w1_ref, b1_ref, w2_ref, b2_ref, gw_ref, u_ref,
                 alpha_ref, gum_ref, pos_ref, inv_ref, starts_ref,
                 ntiles_ref, rws_ref):
    f32 = jnp.float32
    x = x_ref[...]
    T = x.shape[0]

    a1 = lax.dot_general(x, w1_ref[...], (((1,), (1,)), ((), ()))) + b1_ref[...]
    h = a1 / (1.0 + jnp.exp(-a1))
    zlog = lax.dot_general(h, w2_ref[...], (((1,), (1,)), ((), ()))) + b2_ref[...]

    g = zlog + gum_ref[...]
    gmax = jnp.max(g, axis=1, keepdims=True)
    iotz = lax.broadcasted_iota(jnp.int32, (T, _NZ), 1)
    zidx = jnp.min(jnp.where(g == gmax, iotz, _NZ), axis=1, keepdims=True)
    zoh = (iotz == zidx).astype(f32)
    zbias = lax.dot_general(zoh, u_ref[...], (((1,), (0,)), ((), ())))

    logits = lax.dot_general(x, gw_ref[...], (((1,), (1,)), ((), ())))
    logits = logits + alpha_ref[0, 0] * zbias
    lmax = jnp.max(logits, axis=1, keepdims=True)
    el = jnp.exp(logits - lmax)
    probs = el / jnp.sum(el, axis=1, keepdims=True)
    pmax = jnp.max(probs, axis=1, keepdims=True)
    iote = lax.broadcasted_iota(jnp.int32, (T, _E), 1)
    sel = jnp.min(jnp.where(probs == pmax, iote, _E), axis=1, keepdims=True)
    sel_oh = (iote == sel).astype(f32)

    ones_row = jnp.ones((1, T), f32)
    counts = lax.dot_general(ones_row, sel_oh, (((1,), (0,)), ((), ())))
    ptiles = jnp.floor((counts + (_TILE - 1)) * (1.0 / _TILE))
    pc = ptiles * _TILE
    ia = lax.broadcasted_iota(jnp.int32, (_E, _E), 0)
    ib = lax.broadcasted_iota(jnp.int32, (_E, _E), 1)
    tri = (ia < ib).astype(f32)
    pstarts = lax.dot_general(pc, tri, (((1,), (0,)), ((), ())))

    eqf = lax.dot_general(sel_oh, sel_oh, (((1,), (1,)), ((), ())))
    it0 = lax.broadcasted_iota(jnp.int32, (T, T), 0)
    it1 = lax.broadcasted_iota(jnp.int32, (T, T), 1)
    lt = (it1 < it0).astype(f32)
    rank_col = jnp.sum(eqf * lt, axis=1, keepdims=True)

    pstart_sel = lax.dot_general(sel_oh, pstarts, (((1,), (1,)), ((), ())))
    pos_col = pstart_sel + rank_col
    eye = (it0 == it1).astype(f32)
    pos_row = lax.dot_general(ones_row, eye * pos_col, (((1,), (0,)), ((), ())))

    mp = (lax.broadcasted_iota(jnp.int32, (_P, T), 0).astype(f32)
          == pos_row).astype(f32)
    ar_col = lax.broadcasted_iota(jnp.int32, (T, 1), 0).astype(f32)
    inv_col = lax.dot_general(mp, ar_col, (((1,), (0,)), ((), ())))
    rws_col = lax.dot_general(mp, pmax, (((1,), (0,)), ((), ())))

    pos_ref[...] = pos_row.astype(jnp.int32)
    inv_ref[...] = inv_col.astype(jnp.int32)
    starts_ref[...] = pstarts.astype(jnp.int32)
    ntiles_ref[...] = ptiles.astype(jnp.int32)
    rws_ref[...] = rws_col


def _router(flat, W1, b1r, W2, b2r, gate_w, U, alpha_r, gumbel):
    T = flat.shape[0]
    return pl.pallas_call(
        _router_body,
        out_shape=(
            jax.ShapeDtypeStruct((1, T), jnp.int32),
            jax.ShapeDtypeStruct((_P, 1), jnp.int32),
            jax.ShapeDtypeStruct((1, _E), jnp.int32),
            jax.ShapeDtypeStruct((1, _E), jnp.int32),
            jax.ShapeDtypeStruct((_P, 1), jnp.float32),
        ),
        in_specs=[
            pl.BlockSpec(memory_space=pltpu.VMEM),
            pl.BlockSpec(memory_space=pltpu.VMEM),
            pl.BlockSpec(memory_space=pltpu.VMEM),
            pl.BlockSpec(memory_space=pltpu.VMEM),
            pl.BlockSpec(memory_space=pltpu.VMEM),
            pl.BlockSpec(memory_space=pltpu.VMEM),
            pl.BlockSpec(memory_space=pltpu.VMEM),
            pl.BlockSpec(memory_space=pltpu.SMEM),
            pl.BlockSpec(memory_space=pltpu.VMEM),
        ],
        out_specs=(
            pl.BlockSpec(memory_space=pltpu.VMEM),
            pl.BlockSpec(memory_space=pltpu.VMEM),
            pl.BlockSpec(memory_space=pltpu.VMEM),
            pl.BlockSpec(memory_space=pltpu.VMEM),
            pl.BlockSpec(memory_space=pltpu.VMEM),
        ),
    )(flat, W1, b1r, W2, b2r, gate_w, U, alpha_r, gumbel)


def _sc_gather(idx, table, n_rows):
    d = table.shape[1]
    rpt = n_rows // _NW
    mesh = plsc.VectorSubcoreMesh(core_axis_name="c", subcore_axis_name="s")

    @functools.partial(
        pl.kernel,
        out_type=jax.ShapeDtypeStruct((n_rows, d), table.dtype),
        mesh=mesh,
        scratch_types=[
            pltpu.VMEM((rpt,), jnp.int32),
            pltpu.VMEM((rpt, d), table.dtype),
            pltpu.SemaphoreType.DMA,
        ],
    )
    def gather_k(idx_hbm, table_hbm, out_hbm, idx_v, rows_v, sem):
        wid = lax.axis_index("s") * _NC + lax.axis_index("c")
        base = wid * rpt
        pltpu.sync_copy(idx_hbm.at[pl.ds(base, rpt)], idx_v)
        copies = [
            pltpu.async_copy(
                table_hbm.at[idx_v.at[pl.ds(j * 8, 8)]],
                rows_v.at[pl.ds(j * 8, 8)], sem)
            for j in range(rpt // 8)
        ]
        for c in copies:
            c.wait()
        pltpu.sync_copy(rows_v, out_hbm.at[pl.ds(base, rpt)])

    return gather_k(idx, table)


def _expert_body(starts_ref, ntiles_ref, flat_ref, inv_ref, wg_ref, wu_ref,
                 wd_ref, rws_ref, out_ref):
    e = pl.program_id(0)
    start = starts_ref[0, e]
    nt = ntiles_ref[0, e]
    wg = wg_ref[0]
    wu = wu_ref[0]
    wd = wd_ref[0]
    flat = flat_ref[...]
    T = flat.shape[0]

    def tile_body(i, carry):
        off = pl.multiple_of(start + i * _TILE, _TILE)
        i8 = inv_ref[pl.ds(off, _TILE), :]
        g8 = (lax.broadcasted_iota(jnp.int32, (_TILE, T), 1)
              == i8).astype(jnp.float32)
        x8 = lax.dot_general(g8, flat, (((1,), (0,)), ((), ())))
        gg = lax.dot_general(x8, wg, (((1,), (1,)), ((), ())))
        uu = lax.dot_general(x8, wu, (((1,), (1,)), ((), ())))
        hh = gg / (1.0 + jnp.exp(-gg)) * uu
        yy = lax.dot_general(hh, wd, (((1,), (1,)), ((), ())))
        out_ref[pl.ds(off, _TILE), :] = yy * rws_ref[pl.ds(off, _TILE), :]
        return carry

    lax.fori_loop(0, nt, tile_body, 0)


def _experts(flat, inv, w_gate, w_up, w_down, rws, pstarts, ntiles):
    dff, d = w_gate.shape[1], w_gate.shape[2]
    t = flat.shape[0]
    return pl.pallas_call(
        _expert_body,
        grid=(_E,),
        out_shape=jax.ShapeDtypeStruct((_P, d), jnp.float32),
        in_specs=[
            pl.BlockSpec(memory_space=pltpu.SMEM),
            pl.BlockSpec(memory_space=pltpu.SMEM),
            pl.BlockSpec((t, d), lambda e: (0, 0)),
            pl.BlockSpec((_P, 1), lambda e: (0, 0)),
            pl.BlockSpec((1, dff, d), lambda e: (e, 0, 0)),
            pl.BlockSpec((1, dff, d), lambda e: (e, 0, 0)),
            pl.BlockSpec((1, d, dff), lambda e: (e, 0, 0)),
            pl.BlockSpec((_P, 1), lambda e: (0, 0)),
        ],
        out_specs=pl.BlockSpec((_P, d), lambda e: (0, 0)),
        compiler_params=pltpu.CompilerParams(
            dimension_semantics=("arbitrary",),
        ),
    )(pstarts, ntiles, flat, inv, w_gate, w_up, w_down, rws)


def kernel(hidden_states, W1, b1, W2, b2, gate_w, U, alpha, w_gate, w_up,
           w_down, gumbel):
    bq, sq, d = hidden_states.shape
    flat = hidden_states.reshape(-1, d)
    b1r = b1.reshape(1, -1)
    b2r = b2.reshape(1, -1)
    alpha_r = jnp.asarray(alpha, jnp.float32).reshape(1, 1)

    pos, inv, pstarts, ntiles, rws = _router(
        flat, W1, b1r, W2, b2r, gate_w, U, alpha_r, gumbel)
    out_sorted = _experts(flat, inv, w_gate, w_up, w_down, rws, pstarts, ntiles)
    out = _sc_gather(pos.reshape(-1), out_sorted, flat.shape[0])
    return out.reshape(bq, sq, d)


# speedup vs baseline: 1.2328x; 1.0059x over previous
import functools

import jax
import jax.numpy as jnp
from jax import lax
from jax.experimental import pallas as pl
from jax.experimental.pallas import tpu as pltpu
from jax.experimental.pallas import tpu_sc as plsc

_E = 64
_NZ = 8
_TILE = 8
_P = 768
_NC = 2
_NS = 16
_NW = _NC * _NS


def _router_body(x_ref, w1_ref, b1_ref, w2_ref, b2_ref, gw_ref, u_ref,
                 alpha_ref, gum_ref, pos_ref, mp_ref, starts_ref,
                 ntiles_ref, rws_ref):
    f32 = jnp.float32
    x = x_ref[...]
    T = x.shape[0]

    a1 = lax.dot_general(x, w1_ref[...], (((1,), (1,)), ((), ()))) + b1_ref[...]
    h = a1 / (1.0 + jnp.exp(-a1))
    zlog = lax.dot_general(h, w2_ref[...], (((1,), (1,)), ((), ()))) + b2_ref[...]

    g = zlog + gum_ref[...]
    gmax = jnp.max(g, axis=1, keepdims=True)
    iotz = lax.broadcasted_iota(jnp.int32, (T, _NZ), 1)
    zidx = jnp.min(jnp.where(g == gmax, iotz, _NZ), axis=1, keepdims=True)
    zoh = (iotz == zidx).astype(f32)
    zbias = lax.dot_general(zoh, u_ref[...], (((1,), (0,)), ((), ())))

    logits = lax.dot_general(x, gw_ref[...], (((1,), (1,)), ((), ())))
    logits = logits + alpha_ref[0, 0] * zbias
    lmax = jnp.max(logits, axis=1, keepdims=True)
    el = jnp.exp(logits - lmax)
    probs = el / jnp.sum(el, axis=1, keepdims=True)
    pmax = jnp.max(probs, axis=1, keepdims=True)
    iote = lax.broadcasted_iota(jnp.int32, (T, _E), 1)
    sel = jnp.min(jnp.where(probs == pmax, iote, _E), axis=1, keepdims=True)
    sel_oh = (iote == sel).astype(f32)

    ones_row = jnp.ones((1, T), f32)
    counts = lax.dot_general(ones_row, sel_oh, (((1,), (0,)), ((), ())))
    ptiles = jnp.floor((counts + (_TILE - 1)) * (1.0 / _TILE))
    pc = ptiles * _TILE
    ia = lax.broadcasted_iota(jnp.int32, (_E, _E), 0)
    ib = lax.broadcasted_iota(jnp.int32, (_E, _E), 1)
    tri = (ia < ib).astype(f32)
    pstarts = lax.dot_general(pc, tri, (((1,), (0,)), ((), ())))

    eqf = lax.dot_general(sel_oh, sel_oh, (((1,), (1,)), ((), ())))
    it0 = lax.broadcasted_iota(jnp.int32, (T, T), 0)
    it1 = lax.broadcasted_iota(jnp.int32, (T, T), 1)
    lt = (it1 < it0).astype(f32)
    rank_col = jnp.sum(eqf * lt, axis=1, keepdims=True)

    pstart_sel = lax.dot_general(sel_oh, pstarts, (((1,), (1,)), ((), ())))
    pos_col = pstart_sel + rank_col
    eye = (it0 == it1).astype(f32)
    pos_row = lax.dot_general(ones_row, eye * pos_col, (((1,), (0,)), ((), ())))

    mp = (lax.broadcasted_iota(jnp.int32, (_P, T), 0).astype(f32)
          == pos_row).astype(f32)
    rws_col = lax.dot_general(mp, pmax, (((1,), (0,)), ((), ())))

    pos_ref[...] = pos_row.astype(jnp.int32)
    mp_ref[...] = mp
    starts_ref[...] = pstarts.astype(jnp.int32)
    ntiles_ref[...] = ptiles.astype(jnp.int32)
    rws_ref[...] = rws_col


def _router(flat, W1, b1r, W2, b2r, gate_w, U, alpha_r, gumbel):
    T = flat.shape[0]
    return pl.pallas_call(
        _router_body,
        out_shape=(
            jax.ShapeDtypeStruct((1, T), jnp.int32),
            jax.ShapeDtypeStruct((_P, T), jnp.float32),
            jax.ShapeDtypeStruct((1, _E), jnp.int32),
            jax.ShapeDtypeStruct((1, _E), jnp.int32),
            jax.ShapeDtypeStruct((_P, 1), jnp.float32),
        ),
        in_specs=[
            pl.BlockSpec(memory_space=pltpu.VMEM),
            pl.BlockSpec(memory_space=pltpu.VMEM),
            pl.BlockSpec(memory_space=pltpu.VMEM),
            pl.BlockSpec(memory_space=pltpu.VMEM),
            pl.BlockSpec(memory_space=pltpu.VMEM),
            pl.BlockSpec(memory_space=pltpu.VMEM),
            pl.BlockSpec(memory_space=pltpu.VMEM),
            pl.BlockSpec(memory_space=pltpu.SMEM),
            pl.BlockSpec(memory_space=pltpu.VMEM),
        ],
        out_specs=(
            pl.BlockSpec(memory_space=pltpu.VMEM),
            pl.BlockSpec(memory_space=pltpu.VMEM),
            pl.BlockSpec(memory_space=pltpu.VMEM),
            pl.BlockSpec(memory_space=pltpu.VMEM),
            pl.BlockSpec(memory_space=pltpu.VMEM),
        ),
    )(flat, W1, b1r, W2, b2r, gate_w, U, alpha_r, gumbel)


def _sc_gather(idx, table, n_rows):
    d = table.shape[1]
    rpt = n_rows // _NW
    mesh = plsc.VectorSubcoreMesh(core_axis_name="c", subcore_axis_name="s")

    @functools.partial(
        pl.kernel,
        out_type=jax.ShapeDtypeStruct((n_rows, d), table.dtype),
        mesh=mesh,
        scratch_types=[
            pltpu.VMEM((rpt,), jnp.int32),
            pltpu.VMEM((rpt, d), table.dtype),
            pltpu.SemaphoreType.DMA,
        ],
    )
    def gather_k(idx_hbm, table_hbm, out_hbm, idx_v, rows_v, sem):
        wid = lax.axis_index("s") * _NC + lax.axis_index("c")
        base = wid * rpt
        pltpu.sync_copy(idx_hbm.at[pl.ds(base, rpt)], idx_v)
        copies = [
            pltpu.async_copy(
                table_hbm.at[idx_v.at[pl.ds(j * 8, 8)]],
                rows_v.at[pl.ds(j * 8, 8)], sem)
            for j in range(rpt // 8)
        ]
        for c in copies:
            c.wait()
        pltpu.sync_copy(rows_v, out_hbm.at[pl.ds(base, rpt)])

    return gather_k(idx, table)


def _expert_body(starts_ref, ntiles_ref, flat_ref, mp_ref, wg_ref, wu_ref,
                 wd_ref, rws_ref, out_ref):
    e = pl.program_id(0)
    start = starts_ref[0, e]
    nt = ntiles_ref[0, e]
    wg = wg_ref[0]
    wu = wu_ref[0]
    wd = wd_ref[0]
    flat = flat_ref[...]

    def tile_body(i, carry):
        off = pl.multiple_of(start + i * _TILE, _TILE)
        g8 = mp_ref[pl.ds(off, _TILE), :]
        x8 = lax.dot_general(g8, flat, (((1,), (0,)), ((), ())))
        gg = lax.dot_general(x8, wg, (((1,), (1,)), ((), ())))
        uu = lax.dot_general(x8, wu, (((1,), (1,)), ((), ())))
        hh = gg / (1.0 + jnp.exp(-gg)) * uu
        yy = lax.dot_general(hh, wd, (((1,), (1,)), ((), ())))
        out_ref[pl.ds(off, _TILE), :] = yy * rws_ref[pl.ds(off, _TILE), :]
        return carry

    lax.fori_loop(0, nt, tile_body, 0)


def _experts(flat, mp, w_gate, w_up, w_down, rws, pstarts, ntiles):
    dff, d = w_gate.shape[1], w_gate.shape[2]
    t = flat.shape[0]
    return pl.pallas_call(
        _expert_body,
        grid=(_E,),
        out_shape=jax.ShapeDtypeStruct((_P, d), jnp.float32),
        in_specs=[
            pl.BlockSpec(memory_space=pltpu.SMEM),
            pl.BlockSpec(memory_space=pltpu.SMEM),
            pl.BlockSpec((t, d), lambda e: (0, 0)),
            pl.BlockSpec((_P, t), lambda e: (0, 0)),
            pl.BlockSpec((1, dff, d), lambda e: (e, 0, 0)),
            pl.BlockSpec((1, dff, d), lambda e: (e, 0, 0)),
            pl.BlockSpec((1, d, dff), lambda e: (e, 0, 0)),
            pl.BlockSpec((_P, 1), lambda e: (0, 0)),
        ],
        out_specs=pl.BlockSpec((_P, d), lambda e: (0, 0)),
        compiler_params=pltpu.CompilerParams(
            dimension_semantics=("arbitrary",),
        ),
    )(pstarts, ntiles, flat, mp, w_gate, w_up, w_down, rws)


def kernel(hidden_states, W1, b1, W2, b2, gate_w, U, alpha, w_gate, w_up,
           w_down, gumbel):
    bq, sq, d = hidden_states.shape
    flat = hidden_states.reshape(-1, d)
    b1r = b1.reshape(1, -1)
    b2r = b2.reshape(1, -1)
    alpha_r = jnp.asarray(alpha, jnp.float32).reshape(1, 1)

    pos, mp, pstarts, ntiles, rws = _router(
        flat, W1, b1r, W2, b2r, gate_w, U, alpha_r, gumbel)
    out_sorted = _experts(flat, mp, w_gate, w_up, w_down, rws, pstarts, ntiles)
    out = _sc_gather(pos.reshape(-1), out_sorted, flat.shape[0])
    return out.reshape(bq, sq, d)


# speedup vs baseline: 1.2928x; 1.0486x over previous
import functools

import jax
import jax.numpy as jnp
from jax import lax
from jax.experimental import pallas as pl
from jax.experimental.pallas import tpu as pltpu
from jax.experimental.pallas import tpu_sc as plsc

_E = 64
_NZ = 8
_TILE = 8
_P = 768
_NC = 2
_NS = 16
_NW = _NC * _NS


def _router_body(x_ref, w1_ref, b1_ref, w2_ref, b2_ref, gw_ref, u_ref,
                 alpha_ref, gum_ref, pos_ref, xs_ref, starts_ref,
                 ntiles_ref, rws_ref):
    f32 = jnp.float32
    x = x_ref[...]
    T = x.shape[0]

    a1 = lax.dot_general(x, w1_ref[...], (((1,), (1,)), ((), ()))) + b1_ref[...]
    h = a1 / (1.0 + jnp.exp(-a1))
    zlog = lax.dot_general(h, w2_ref[...], (((1,), (1,)), ((), ()))) + b2_ref[...]

    g = zlog + gum_ref[...]
    gmax = jnp.max(g, axis=1, keepdims=True)
    iotz = lax.broadcasted_iota(jnp.int32, (T, _NZ), 1)
    zidx = jnp.min(jnp.where(g == gmax, iotz, _NZ), axis=1, keepdims=True)
    zoh = (iotz == zidx).astype(f32)
    zbias = lax.dot_general(zoh, u_ref[...], (((1,), (0,)), ((), ())))

    logits = lax.dot_general(x, gw_ref[...], (((1,), (1,)), ((), ())))
    logits = logits + alpha_ref[0, 0] * zbias
    lmax = jnp.max(logits, axis=1, keepdims=True)
    el = jnp.exp(logits - lmax)
    probs = el / jnp.sum(el, axis=1, keepdims=True)
    pmax = jnp.max(probs, axis=1, keepdims=True)
    iote = lax.broadcasted_iota(jnp.int32, (T, _E), 1)
    sel = jnp.min(jnp.where(probs == pmax, iote, _E), axis=1, keepdims=True)
    sel_oh = (iote == sel).astype(f32)

    ones_row = jnp.ones((1, T), f32)
    counts = lax.dot_general(ones_row, sel_oh, (((1,), (0,)), ((), ())))
    ptiles = jnp.floor((counts + (_TILE - 1)) * (1.0 / _TILE))
    pc = ptiles * _TILE
    ia = lax.broadcasted_iota(jnp.int32, (_E, _E), 0)
    ib = lax.broadcasted_iota(jnp.int32, (_E, _E), 1)
    tri = (ia < ib).astype(f32)
    pstarts = lax.dot_general(pc, tri, (((1,), (0,)), ((), ())))

    eqf = lax.dot_general(sel_oh, sel_oh, (((1,), (1,)), ((), ())))
    it0 = lax.broadcasted_iota(jnp.int32, (T, T), 0)
    it1 = lax.broadcasted_iota(jnp.int32, (T, T), 1)
    lt = (it1 < it0).astype(f32)
    rank_col = jnp.sum(eqf * lt, axis=1, keepdims=True)

    pstart_sel = lax.dot_general(sel_oh, pstarts, (((1,), (1,)), ((), ())))
    pos_col = pstart_sel + rank_col
    eye = (it0 == it1).astype(f32)
    pos_row = lax.dot_general(ones_row, eye * pos_col, (((1,), (0,)), ((), ())))

    mp = (lax.broadcasted_iota(jnp.int32, (_P, T), 0).astype(f32)
          == pos_row).astype(f32)
    rws_col = lax.dot_general(mp, pmax, (((1,), (0,)), ((), ())))

    pos_ref[...] = pos_row.astype(jnp.int32)
    xs_ref[...] = lax.dot_general(mp, x, (((1,), (0,)), ((), ())))
    starts_ref[...] = pstarts.astype(jnp.int32)
    ntiles_ref[...] = ptiles.astype(jnp.int32)
    rws_ref[...] = rws_col


def _router(flat, W1, b1r, W2, b2r, gate_w, U, alpha_r, gumbel):
    T = flat.shape[0]
    return pl.pallas_call(
        _router_body,
        out_shape=(
            jax.ShapeDtypeStruct((1, T), jnp.int32),
            jax.ShapeDtypeStruct((_P, flat.shape[1]), jnp.float32),
            jax.ShapeDtypeStruct((1, _E), jnp.int32),
            jax.ShapeDtypeStruct((1, _E), jnp.int32),
            jax.ShapeDtypeStruct((_P, 1), jnp.float32),
        ),
        in_specs=[
            pl.BlockSpec(memory_space=pltpu.VMEM),
            pl.BlockSpec(memory_space=pltpu.VMEM),
            pl.BlockSpec(memory_space=pltpu.VMEM),
            pl.BlockSpec(memory_space=pltpu.VMEM),
            pl.BlockSpec(memory_space=pltpu.VMEM),
            pl.BlockSpec(memory_space=pltpu.VMEM),
            pl.BlockSpec(memory_space=pltpu.VMEM),
            pl.BlockSpec(memory_space=pltpu.SMEM),
            pl.BlockSpec(memory_space=pltpu.VMEM),
        ],
        out_specs=(
            pl.BlockSpec(memory_space=pltpu.VMEM),
            pl.BlockSpec(memory_space=pltpu.VMEM),
            pl.BlockSpec(memory_space=pltpu.VMEM),
            pl.BlockSpec(memory_space=pltpu.VMEM),
            pl.BlockSpec(memory_space=pltpu.VMEM),
        ),
    )(flat, W1, b1r, W2, b2r, gate_w, U, alpha_r, gumbel)


def _sc_gather(idx, table, n_rows):
    d = table.shape[1]
    rpt = n_rows // _NW
    mesh = plsc.VectorSubcoreMesh(core_axis_name="c", subcore_axis_name="s")

    @functools.partial(
        pl.kernel,
        out_type=jax.ShapeDtypeStruct((n_rows, d), table.dtype),
        mesh=mesh,
        scratch_types=[
            pltpu.VMEM((rpt,), jnp.int32),
            pltpu.VMEM((rpt, d), table.dtype),
            pltpu.SemaphoreType.DMA,
        ],
    )
    def gather_k(idx_hbm, table_hbm, out_hbm, idx_v, rows_v, sem):
        wid = lax.axis_index("s") * _NC + lax.axis_index("c")
        base = wid * rpt
        pltpu.sync_copy(idx_hbm.at[pl.ds(base, rpt)], idx_v)
        copies = [
            pltpu.async_copy(
                table_hbm.at[idx_v.at[pl.ds(j * 8, 8)]],
                rows_v.at[pl.ds(j * 8, 8)], sem)
            for j in range(rpt // 8)
        ]
        for c in copies:
            c.wait()
        pltpu.sync_copy(rows_v, out_hbm.at[pl.ds(base, rpt)])

    return gather_k(idx, table)


def _expert_body(starts_ref, ntiles_ref, xs_ref, wg_ref, wu_ref,
                 wd_ref, rws_ref, out_ref):
    e = pl.program_id(0)
    start = starts_ref[0, e]
    nt = ntiles_ref[0, e]
    wg = wg_ref[0]
    wu = wu_ref[0]
    wd = wd_ref[0]

    def tile_body(i, carry):
        off = pl.multiple_of(start + i * _TILE, _TILE)
        x8 = xs_ref[pl.ds(off, _TILE), :]
        gg = lax.dot_general(x8, wg, (((1,), (1,)), ((), ())))
        uu = lax.dot_general(x8, wu, (((1,), (1,)), ((), ())))
        hh = gg / (1.0 + jnp.exp(-gg)) * uu
        yy = lax.dot_general(hh, wd, (((1,), (1,)), ((), ())))
        out_ref[pl.ds(off, _TILE), :] = yy * rws_ref[pl.ds(off, _TILE), :]
        return carry

    lax.fori_loop(0, nt, tile_body, 0)


def _experts(xs, w_gate, w_up, w_down, rws, pstarts, ntiles):
    dff, d = w_gate.shape[1], w_gate.shape[2]
    return pl.pallas_call(
        _expert_body,
        grid=(_E,),
        out_shape=jax.ShapeDtypeStruct((_P, d), jnp.float32),
        in_specs=[
            pl.BlockSpec(memory_space=pltpu.SMEM),
            pl.BlockSpec(memory_space=pltpu.SMEM),
            pl.BlockSpec((_P, d), lambda e: (0, 0)),
            pl.BlockSpec((1, dff, d), lambda e: (e, 0, 0)),
            pl.BlockSpec((1, dff, d), lambda e: (e, 0, 0)),
            pl.BlockSpec((1, d, dff), lambda e: (e, 0, 0)),
            pl.BlockSpec((_P, 1), lambda e: (0, 0)),
        ],
        out_specs=pl.BlockSpec((_P, d), lambda e: (0, 0)),
        compiler_params=pltpu.CompilerParams(
            dimension_semantics=("arbitrary",),
        ),
    )(pstarts, ntiles, xs, w_gate, w_up, w_down, rws)


def kernel(hidden_states, W1, b1, W2, b2, gate_w, U, alpha, w_gate, w_up,
           w_down, gumbel):
    bq, sq, d = hidden_states.shape
    flat = hidden_states.reshape(-1, d)
    b1r = b1.reshape(1, -1)
    b2r = b2.reshape(1, -1)
    alpha_r = jnp.asarray(alpha, jnp.float32).reshape(1, 1)

    pos, xs, pstarts, ntiles, rws = _router(
        flat, W1, b1r, W2, b2r, gate_w, U, alpha_r, gumbel)
    out_sorted = _experts(xs, w_gate, w_up, w_down, rws, pstarts, ntiles)
    out = _sc_gather(pos.reshape(-1), out_sorted, flat.shape[0])
    return out.reshape(bq, sq, d)
